# baseline (speedup 1.0000x reference)
import functools

import jax
import jax.numpy as jnp
from jax import lax
from jax.experimental import pallas as pl
from jax.experimental.pallas import tpu as pltpu
from jax.experimental.pallas import tpu_sc as plsc

_NC = 2
_NS = 16
_NW = _NC * _NS
_L = 16
_B = 1024
_D = 64
_R = 8
_BPW = _B // _NW


@functools.partial(
    pl.kernel,
    mesh=plsc.VectorSubcoreMesh(core_axis_name="c", subcore_axis_name="s"),
    out_type=jax.ShapeDtypeStruct((_NW, _L), jnp.float32),
    scratch_types=[
        pltpu.VMEM((_BPW,), jnp.int32),
        pltpu.VMEM((_BPW, _D), jnp.float32),
        pltpu.VMEM((_BPW * _R, _D), jnp.float32),
        pltpu.VMEM((_L,), jnp.float32),
        pltpu.SemaphoreType.DMA,
    ],
)
def _center_mse_partials(features_hbm, labels_hbm, centers_hbm, out_hbm,
                         idx_v, feat_v, rows_v, acc_v, sem):
    wid = lax.axis_index("s") * _NC + lax.axis_index("c")
    base = wid * _BPW
    pltpu.sync_copy(labels_hbm.at[pl.ds(base, _BPW)], idx_v)
    lbls = []
    copies = []
    for c in range(_BPW // _L):
        lbl = idx_v[pl.ds(c * _L, _L)]
        for k in range(_L):
            l = lbl[k]
            lbls.append(l)
            grp = pl.multiple_of(lax.bitwise_and(l, jnp.int32(~(_R - 1))), _R)
            copies.append(pltpu.async_copy(
                centers_hbm.at[pl.ds(grp, _R)],
                rows_v.at[pl.ds((c * _L + k) * _R, _R)], sem))
    pltpu.sync_copy(features_hbm.at[pl.ds(base, _BPW)], feat_v)
    for cp in copies:
        cp.wait()
    acc = jnp.zeros((_L,), jnp.float32)
    for i in range(_BPW):
        g = i * _R + lax.bitwise_and(lbls[i], jnp.int32(_R - 1))
        for j in range(_D // _L):
            d = feat_v[i, pl.ds(j * _L, _L)] - rows_v[g, pl.ds(j * _L, _L)]
            acc = acc + d * d
    acc_v[...] = acc
    pltpu.sync_copy(acc_v, out_hbm.at[wid])


def kernel(features, labels, centers):
    partials = _center_mse_partials(
        features, labels.astype(jnp.int32), centers)
    return jnp.sum(partials) / jnp.float32(_B * _D)
